# baseline (speedup 1.0000x reference)
import functools

import jax
import jax.numpy as jnp
from jax.experimental import pallas as pl
from jax.experimental.pallas import tpu as pltpu

N_NODES = 50000
E = 64000
D_IN = 128
H = 2
C = 32
D_INNER = 512
REL_SRC = [0, 1, 0, 2, 1, 2, 3, 2]
REL_DST = [1, 0, 2, 0, 2, 1, 2, 3]

MLP_BLK = 2000


def _mlp_body(x_ref, wz0, bz0, wz1, bz1, wz2, bz2, wv0, bv0, wv1, bv1, wv2, bv2,
              pz_ref, val_ref):
    x = x_ref[...]
    h = jnp.maximum(jnp.dot(x, wz0[...], preferred_element_type=jnp.float32) + bz0[...], 0.0)
    h = jnp.maximum(jnp.dot(h, wz1[...], preferred_element_type=jnp.float32) + bz1[...], 0.0)
    pz_ref[...] = jax.nn.sigmoid(
        jnp.dot(h, wz2[...], preferred_element_type=jnp.float32) + bz2[...])
    h = jnp.maximum(jnp.dot(x, wv0[...], preferred_element_type=jnp.float32) + bv0[...], 0.0)
    h = jnp.maximum(jnp.dot(h, wv1[...], preferred_element_type=jnp.float32) + bv1[...], 0.0)
    val_ref[...] = jnp.dot(h, wv2[...], preferred_element_type=jnp.float32) + bv2[...]


def _mlp_heads(gene, Wz0, bz0, Wz1, bz1, Wz2, bz2, Wv0, bv0, Wv1, bv1, Wv2, bv2):
    n = gene.shape[0]
    grid = n // MLP_BLK
    full = lambda shp: pl.BlockSpec(shp, lambda i: (0,) * len(shp))
    return pl.pallas_call(
        _mlp_body,
        grid=(grid,),
        in_specs=[
            pl.BlockSpec((MLP_BLK, C), lambda i: (i, 0)),
            full((C, D_INNER)), full((1, D_INNER)),
            full((D_INNER, D_INNER)), full((1, D_INNER)),
            full((D_INNER, 1)), full((1, 1)),
            full((C, D_INNER)), full((1, D_INNER)),
            full((D_INNER, D_INNER)), full((1, D_INNER)),
            full((D_INNER, 1)), full((1, 1)),
        ],
        out_specs=[
            pl.BlockSpec((MLP_BLK, 1), lambda i: (i, 0)),
            pl.BlockSpec((MLP_BLK, 1), lambda i: (i, 0)),
        ],
        out_shape=[
            jax.ShapeDtypeStruct((n, 1), jnp.float32),
            jax.ShapeDtypeStruct((n, 1), jnp.float32),
        ],
    )(gene, Wz0, bz0.reshape(1, -1), Wz1, bz1.reshape(1, -1), Wz2,
      bz2.reshape(1, -1), Wv0, bv0.reshape(1, -1), Wv1, bv1.reshape(1, -1),
      Wv2, bv2.reshape(1, -1))


def _gat_rel(x_src, ei, Ws, al_s_vec, al_d_vec, x_dst_unused=None):
    raise NotImplementedError


def _gat_plain(x_src, x_dst, ei, Ws, Wd, a_s, a_d, bias, n_dst):
    src, dst = ei[0], ei[1]
    xs = (x_src @ Ws).reshape(-1, H, C)
    al_s = jnp.sum(xs * a_s[None], axis=-1)
    al_d = jnp.sum((x_dst @ Wd).reshape(-1, H, C) * a_d[None], axis=-1)
    alpha = jax.nn.leaky_relu(al_s[src] + al_d[dst], negative_slope=0.2)
    amax = jax.ops.segment_max(alpha, dst, num_segments=n_dst)
    amax = jnp.where(jnp.isfinite(amax), amax, 0.0)
    ex = jnp.exp(alpha - amax[dst])
    den = jax.ops.segment_sum(ex, dst, num_segments=n_dst)
    att = ex / (den[dst] + 1e-16)
    msg = (xs[src] * att[:, :, None]).reshape(-1, H * C)
    out = jax.ops.segment_sum(msg, dst, num_segments=n_dst).reshape(n_dst, H, C)
    return out.mean(axis=1) + bias


def kernel(x_tad, x_atac, x_gene, x_protein, ei0, ei1, ei2, ei3, ei4, ei5, ei6,
           ei7, W0_src, W0_dst, att0_src, att0_dst, b0, W_src, W_dst, att_src,
           att_dst, b, Wz0, bz0, Wz1, bz1, Wz2, bz2, Wv0, bv0, Wv1, bv1, Wv2, bv2):
    xs = [x_tad, x_atac, x_gene, x_protein]
    eis = [ei0, ei1, ei2, ei3, ei4, ei5, ei6, ei7]
    Ns = [x.shape[0] for x in xs]
    for l in range(4):
        if l == 0:
            Ws_l, Wd_l, as_l, ad_l, b_l = W0_src, W0_dst, att0_src, att0_dst, b0
        else:
            Ws_l, Wd_l, as_l, ad_l, b_l = (W_src[l - 1], W_dst[l - 1],
                                           att_src[l - 1], att_dst[l - 1], b[l - 1])
        new = [jnp.zeros((n, C), dtype=jnp.float32) for n in Ns]
        for r in range(8):
            s, d = REL_SRC[r], REL_DST[r]
            new[d] = new[d] + _gat_plain(xs[s], xs[d], eis[r], Ws_l[r], Wd_l[r],
                                         as_l[r], ad_l[r], b_l[r], Ns[d])
        xs = [jax.nn.relu(v) for v in new]
    gene = xs[2]
    p_zero, values = _mlp_heads(gene, Wz0, bz0, Wz1, bz1, Wz2, bz2,
                                Wv0, bv0, Wv1, bv1, Wv2, bv2)
    zeros = jax.random.bernoulli(jax.random.key(1), p_zero).astype(jnp.float32)
    return (p_zero, zeros, values)


# speedup vs baseline: 1.4742x; 1.4742x over previous
import functools

import jax
import jax.numpy as jnp
from jax import lax
from jax.experimental import pallas as pl
from jax.experimental.pallas import tpu as pltpu
from jax.experimental.pallas import tpu_sc as plsc

N_NODES = 50000
E = 64000
D_IN = 128
H = 2
C = 32
D_INNER = 512
REL_SRC = [0, 1, 0, 2, 1, 2, 3, 2]
REL_DST = [1, 0, 2, 0, 2, 1, 2, 3]

N_TILES = 16
NHALF = N_NODES // 2
RH_A = 1568
RH_LAST = NHALF - 15 * RH_A
E_TILE = E // N_TILES
CH = 80
NCH = E_TILE // CH

SRC_COL = {}
DST_COL = {}
for _t in range(4):
    _c = 0
    for _r in range(8):
        if REL_SRC[_r] == _t:
            SRC_COL[_r] = _c
            _c += 2
    for _r in range(8):
        if REL_DST[_r] == _t:
            DST_COL[_r] = _c
            _c += 2

CORE_PLAN = {
    0: [(0, [1, 3]), (1, [0, 5])],
    1: [(2, [2, 4, 6]), (3, [7])],
}

MLP_BLK = 2000
PROJ_BLK = 2000


def _proj_body_l0(x0, x1, x2, x3, w, va0, va1, va2, va3, *outs):
    xs = (x0[...], x1[...], x2[...], x3[...])
    vas = (va0, va1, va2, va3)
    for r in range(8):
        outs[r][...] = jnp.dot(xs[REL_SRC[r]], w[r], preferred_element_type=jnp.float32)
    for t in range(4):
        outs[8 + t][...] = jnp.dot(xs[t], vas[t][...], preferred_element_type=jnp.float32)


def _proj_body_l(m0, m1, m2, m3, bsum, w, va0, va1, va2, va3, *outs):
    ms = (m0[...], m1[...], m2[...], m3[...])
    bs = bsum[...]
    vas = (va0, va1, va2, va3)
    xs = [jnp.maximum(ms[t] + bs[t], 0.0) for t in range(4)]
    for r in range(8):
        outs[r][...] = jnp.dot(xs[REL_SRC[r]], w[r], preferred_element_type=jnp.float32)
    for t in range(4):
        outs[8 + t][...] = jnp.dot(xs[t], vas[t][...], preferred_element_type=jnp.float32)


def _proj_call(xs_or_msgs, w_stack, vas, bsum):
    n = N_NODES
    d = w_stack.shape[1]
    grid = n // PROJ_BLK
    full = lambda shp: pl.BlockSpec(shp, lambda i: (0,) * len(shp))
    row = lambda c: pl.BlockSpec((PROJ_BLK, c), lambda i: (i, 0))
    if bsum is None:
        body = _proj_body_l0
        ins = list(xs_or_msgs) + [w_stack] + list(vas)
        in_specs = [row(d)] * 4 + [full((8, d, H * C))] + [full((d, 16))] * 4
    else:
        body = _proj_body_l
        ins = list(xs_or_msgs) + [bsum, w_stack] + list(vas)
        in_specs = ([row(C)] * 4 + [full((4, C)), full((8, d, H * C))]
                    + [full((d, 16))] * 4)
    out_specs = [row(H * C)] * 8 + [row(16)] * 4
    out_shape = ([jax.ShapeDtypeStruct((n, H * C), jnp.float32)] * 8
                 + [jax.ShapeDtypeStruct((n, 16), jnp.float32)] * 4)
    res = pl.pallas_call(
        body, grid=(grid,), in_specs=in_specs, out_specs=out_specs,
        out_shape=out_shape,
    )(*ins)
    return res[:8], res[8:]


def _iota16():
    return lax.iota(jnp.int32, 16)


def _full16(v):
    return jnp.full((16,), v, jnp.int32)


def _edge_body(*refs):
    xs_refs = refs[0:8]
    al_refs = refs[8:12]
    src_refs = refs[12:20]
    dst_refs = refs[20:28]
    z32h, z2h = refs[28:30]
    out_refs = refs[30:34]
    (acc, den, srcc, dstc, dstadj, alsv, aldv, exall, exsrc, denv, xsv,
     msgv) = refs[34:]

    cid = lax.axis_index("c")
    tid = lax.axis_index("s")
    rbase = tid * RH_A
    ebase = tid * E_TILE
    iota = _iota16()

    def _zx(k, carry):
        f = iota + 16 * k
        plsc.store_scatter(exsrc, [lax.shift_right_logical(f, 3),
                                   lax.bitwise_and(f, 7)],
                           jnp.zeros((16,), jnp.float32))
        return carry
    lax.fori_loop(0, CH * 8 // 16, _zx, 0)

    def split_copy(src_fn, dst_fn):
        @pl.when(tid < 15)
        def _():
            pltpu.sync_copy(src_fn(rbase, RH_A), dst_fn(rbase, RH_A))

        @pl.when(tid == 15)
        def _():
            pltpu.sync_copy(src_fn(rbase, RH_LAST), dst_fn(rbase, RH_LAST))

    def zero_acc():
        split_copy(lambda o, s: z32h.at[pl.ds(o, s), :],
                   lambda o, s: acc.at[pl.ds(o, s), :])

    def zero_den():
        split_copy(lambda o, s: z2h.at[pl.ds(o, s), :],
                   lambda o, s: den.at[pl.ds(o, s), :])

    def clamp_dst(h0):
        def grp(g, carry):
            e16 = iota + 16 * g
            d = plsc.load_gather(dstc, [e16])
            cl = jnp.minimum(jnp.maximum(d - h0, 0), NHALF - 1)
            dstadj[pl.ds(g * 16, 16)] = cl
            return carry
        lax.fori_loop(0, CH // 16, grp, 0)

    def pass_a(r, h0):
        al_s = al_refs[REL_SRC[r]]
        al_d = al_refs[REL_DST[r]]
        scol = SRC_COL[r]
        dcol = DST_COL[r]

        def chunk(i, carry):
            eb = ebase + i * CH
            pltpu.sync_copy(src_refs[r].at[pl.ds(eb, CH)], srcc)
            pltpu.sync_copy(dst_refs[r].at[pl.ds(eb, CH)], dstc)
            pltpu.sync_copy(al_s.at[srcc], alsv)
            pltpu.sync_copy(al_d.at[dstc], aldv)
            clamp_dst(h0)
            exc = exall.at[pl.ds(i * CH, CH), :]

            def pair(j, carry2):
                f = iota + 16 * j
                row = lax.shift_right_logical(f, 1)
                col = lax.bitwise_and(f, 1)
                a = (plsc.load_gather(alsv, [row, scol + col])
                     + plsc.load_gather(aldv, [row, dcol + col]))
                a = jnp.where(a >= 0.0, a, 0.2 * a)
                e = jnp.exp(a)
                plsc.store_scatter(exc, [row, col], e)
                d = plsc.load_gather(dstc, [row])
                inh = jnp.logical_and(d >= h0, d < h0 + NHALF)
                em = jnp.where(inh, e, 0.0)
                plsc.store_scatter(exsrc, [row, col], em)
                return carry2

            lax.fori_loop(0, CH * 2 // 16, pair, 0)
            pltpu.sync_copy(exsrc, den.at[dstadj], add=True)
            return carry

        lax.fori_loop(0, NCH, chunk, 0)

    def pass_b(r, h0):
        xs_r = xs_refs[r]

        def chunk(i, carry):
            eb = ebase + i * CH
            pltpu.sync_copy(src_refs[r].at[pl.ds(eb, CH)], srcc)
            pltpu.sync_copy(dst_refs[r].at[pl.ds(eb, CH)], dstc)
            clamp_dst(h0)
            pltpu.sync_copy(den.at[dstadj], denv)
            pltpu.sync_copy(xs_r.at[srcc], xsv)
            exc = exall.at[pl.ds(i * CH, CH), :]

            def grp(g, carry2):
                e16 = iota + 16 * g
                d = plsc.load_gather(dstc, [e16])
                inh = jnp.logical_and(d >= h0, d < h0 + NHALF)
                sel = jnp.where(inh, 0.5, 0.0)
                a0 = (plsc.load_gather(exc, [e16, _full16(0)])
                      / (plsc.load_gather(denv, [e16, _full16(0)]) + 1e-16))
                a1 = (plsc.load_gather(exc, [e16, _full16(1)])
                      / (plsc.load_gather(denv, [e16, _full16(1)]) + 1e-16))
                a0 = a0 * sel
                a1 = a1 * sel

                def feat(j, carry3):
                    jf = jnp.full((16,), j, jnp.int32)
                    m = (plsc.load_gather(xsv, [e16, jf]) * a0
                         + plsc.load_gather(xsv, [e16, jf + C]) * a1)
                    plsc.store_scatter(msgv, [e16, jf], m)
                    return carry3

                lax.fori_loop(0, C, feat, 0)
                return carry2

            lax.fori_loop(0, CH // 16, grp, 0)
            pltpu.sync_copy(msgv, acc.at[dstadj], add=True)
            return carry

        lax.fori_loop(0, NCH, chunk, 0)

    def run_plan(plan):
        for t, rels in plan:
            for half in range(2):
                h0 = half * NHALF
                zero_acc()
                for r in rels:
                    zero_den()
                    plsc.subcore_barrier()
                    pass_a(r, h0)
                    plsc.subcore_barrier()
                    pass_b(r, h0)
                    plsc.subcore_barrier()
                split_copy(
                    lambda o, s: acc.at[pl.ds(o, s), :],
                    lambda o, s: out_refs[t].at[pl.ds(h0 + o, s), :])
                plsc.subcore_barrier()

    @pl.when(cid == 0)
    def _():
        run_plan(CORE_PLAN[0])

    @pl.when(cid == 1)
    def _():
        run_plan(CORE_PLAN[1])


def _edge_phase(xs_list, al_list, srcs, dsts, z32h, z2h):
    mesh = plsc.VectorSubcoreMesh(core_axis_name="c", subcore_axis_name="s")
    f = pl.kernel(
        _edge_body,
        out_type=[jax.ShapeDtypeStruct((N_NODES, C), jnp.float32)] * 4,
        mesh=mesh,
        compiler_params=pltpu.CompilerParams(needs_layout_passes=False,
                                             use_tc_tiling_on_sc=False),
        scratch_types=[
            pltpu.VMEM_SHARED((NHALF, C), jnp.float32),
            pltpu.VMEM_SHARED((NHALF, 8), jnp.float32),
            pltpu.VMEM((CH,), jnp.int32),
            pltpu.VMEM((CH,), jnp.int32),
            pltpu.VMEM((CH,), jnp.int32),
            pltpu.VMEM((CH, 16), jnp.float32),
            pltpu.VMEM((CH, 16), jnp.float32),
            pltpu.VMEM((E_TILE, 2), jnp.float32),
            pltpu.VMEM((CH, 8), jnp.float32),
            pltpu.VMEM((CH, 8), jnp.float32),
            pltpu.VMEM((CH, H * C), jnp.float32),
            pltpu.VMEM((CH, C), jnp.float32),
        ],
    )
    return f(*xs_list, *al_list, *srcs, *dsts, z32h, z2h)


def _mlp_body(m_ref, bsum, wz0, bz0, wz1, bz1, wz2, bz2, wv0, bv0, wv1, bv1,
              wv2, bv2, pz_ref, val_ref):
    x = jnp.maximum(m_ref[...] + bsum[...], 0.0)
    h = jnp.maximum(jnp.dot(x, wz0[...], preferred_element_type=jnp.float32) + bz0[...], 0.0)
    h = jnp.maximum(jnp.dot(h, wz1[...], preferred_element_type=jnp.float32) + bz1[...], 0.0)
    pz_ref[...] = jax.nn.sigmoid(
        jnp.dot(h, wz2[...], preferred_element_type=jnp.float32) + bz2[...])
    h = jnp.maximum(jnp.dot(x, wv0[...], preferred_element_type=jnp.float32) + bv0[...], 0.0)
    h = jnp.maximum(jnp.dot(h, wv1[...], preferred_element_type=jnp.float32) + bv1[...], 0.0)
    val_ref[...] = jnp.dot(h, wv2[...], preferred_element_type=jnp.float32) + bv2[...]


def _mlp_heads(msg_gene, bsum_gene, Wz0, bz0, Wz1, bz1, Wz2, bz2,
               Wv0, bv0, Wv1, bv1, Wv2, bv2):
    n = msg_gene.shape[0]
    grid = n // MLP_BLK
    full = lambda shp: pl.BlockSpec(shp, lambda i: (0,) * len(shp))
    return pl.pallas_call(
        _mlp_body,
        grid=(grid,),
        in_specs=[
            pl.BlockSpec((MLP_BLK, C), lambda i: (i, 0)),
            full((1, C)),
            full((C, D_INNER)), full((1, D_INNER)),
            full((D_INNER, D_INNER)), full((1, D_INNER)),
            full((D_INNER, 1)), full((1, 1)),
            full((C, D_INNER)), full((1, D_INNER)),
            full((D_INNER, D_INNER)), full((1, D_INNER)),
            full((D_INNER, 1)), full((1, 1)),
        ],
        out_specs=[
            pl.BlockSpec((MLP_BLK, 1), lambda i: (i, 0)),
            pl.BlockSpec((MLP_BLK, 1), lambda i: (i, 0)),
        ],
        out_shape=[
            jax.ShapeDtypeStruct((n, 1), jnp.float32),
            jax.ShapeDtypeStruct((n, 1), jnp.float32),
        ],
    )(msg_gene, bsum_gene.reshape(1, -1),
      Wz0, bz0.reshape(1, -1), Wz1, bz1.reshape(1, -1), Wz2, bz2.reshape(1, -1),
      Wv0, bv0.reshape(1, -1), Wv1, bv1.reshape(1, -1), Wv2, bv2.reshape(1, -1))


def _fold_logit_vectors(W, att):
    d = W.shape[1]
    return jnp.einsum("rdhc,rhc->rdh", W.reshape(8, d, H, C), att)


def _build_al_weights(W_src_l, W_dst_l, att_src_l, att_dst_l):
    vs = _fold_logit_vectors(W_src_l, att_src_l)
    vd = _fold_logit_vectors(W_dst_l, att_dst_l)
    d = W_src_l.shape[1]
    vas = []
    for t in range(4):
        cols = []
        for r in range(8):
            if REL_SRC[r] == t:
                cols.append(vs[r])
        for r in range(8):
            if REL_DST[r] == t:
                cols.append(vd[r])
        va = jnp.concatenate(cols, axis=1)
        va = jnp.pad(va, ((0, 0), (0, 16 - va.shape[1])))
        vas.append(va)
    return vas


def _bias_sums(b_l):
    return jnp.stack(
        [sum(b_l[r] for r in range(8) if REL_DST[r] == t) for t in range(4)])


def kernel(x_tad, x_atac, x_gene, x_protein, ei0, ei1, ei2, ei3, ei4, ei5, ei6,
           ei7, W0_src, W0_dst, att0_src, att0_dst, b0, W_src, W_dst, att_src,
           att_dst, b, Wz0, bz0, Wz1, bz1, Wz2, bz2, Wv0, bv0, Wv1, bv1, Wv2, bv2):
    xs = [x_tad, x_atac, x_gene, x_protein]
    eis = [ei0, ei1, ei2, ei3, ei4, ei5, ei6, ei7]
    srcs = [e[0] for e in eis]
    dsts = [e[1] for e in eis]
    z32h = jnp.zeros((NHALF, C), jnp.float32)
    z2h = jnp.zeros((NHALF, 8), jnp.float32)

    msgs = None
    for l in range(4):
        if l == 0:
            Ws_l, Wd_l, as_l, ad_l = W0_src, W0_dst, att0_src, att0_dst
            bsum = None
        else:
            Ws_l, Wd_l, as_l, ad_l = (W_src[l - 1], W_dst[l - 1],
                                      att_src[l - 1], att_dst[l - 1])
            bsum = _bias_sums(b[l - 1])
        vas = _build_al_weights(Ws_l, Wd_l, as_l, ad_l)
        src_in = xs if l == 0 else list(msgs)
        xs_proj, al_list = _proj_call(src_in, Ws_l, vas, bsum)
        msgs = _edge_phase(xs_proj, al_list, srcs, dsts, z32h, z2h)

    bsum_last = _bias_sums(b[2])
    p_zero, values = _mlp_heads(msgs[2], bsum_last[2], Wz0, bz0, Wz1, bz1,
                                Wz2, bz2, Wv0, bv0, Wv1, bv1, Wv2, bv2)
    zeros = jax.random.bernoulli(jax.random.key(1), p_zero).astype(jnp.float32)
    return (p_zero, zeros, values)


# speedup vs baseline: 1.7108x; 1.1605x over previous
import functools

import jax
import jax.numpy as jnp
from jax import lax
from jax.experimental import pallas as pl
from jax.experimental.pallas import tpu as pltpu
from jax.experimental.pallas import tpu_sc as plsc

N_NODES = 50000
E = 64000
D_IN = 128
H = 2
C = 32
D_INNER = 512
REL_SRC = [0, 1, 0, 2, 1, 2, 3, 2]
REL_DST = [1, 0, 2, 0, 2, 1, 2, 3]

N_TILES = 16
NHALF = N_NODES // 2
RH_A = 1568
RH_LAST = NHALF - 15 * RH_A
E_TILE = E // N_TILES
CH = 80
NCH = E_TILE // CH

SRC_COL = {}
DST_COL = {}
for _t in range(4):
    _c = 0
    for _r in range(8):
        if REL_SRC[_r] == _t:
            SRC_COL[_r] = _c
            _c += 2
    for _r in range(8):
        if REL_DST[_r] == _t:
            DST_COL[_r] = _c
            _c += 2

CORE_PLAN = {
    0: [(0, [1, 3]), (1, [0, 5])],
    1: [(2, [2, 4, 6]), (3, [7])],
}

MLP_BLK = 2000
PROJ_BLK = 2000


def _proj_body_l0(x0, x1, x2, x3, w, va0, va1, va2, va3, *outs):
    xs = (x0[...], x1[...], x2[...], x3[...])
    vas = (va0, va1, va2, va3)
    for r in range(8):
        outs[r][...] = jnp.dot(xs[REL_SRC[r]], w[r], preferred_element_type=jnp.float32)
    for t in range(4):
        outs[8 + t][...] = jnp.dot(xs[t], vas[t][...], preferred_element_type=jnp.float32)


def _proj_body_l(m0, m1, m2, m3, bsum, w, va0, va1, va2, va3, *outs):
    ms = (m0[...], m1[...], m2[...], m3[...])
    bs = bsum[...]
    vas = (va0, va1, va2, va3)
    xs = [jnp.maximum(ms[t] + bs[t], 0.0) for t in range(4)]
    for r in range(8):
        outs[r][...] = jnp.dot(xs[REL_SRC[r]], w[r], preferred_element_type=jnp.float32)
    for t in range(4):
        outs[8 + t][...] = jnp.dot(xs[t], vas[t][...], preferred_element_type=jnp.float32)


def _proj_call(xs_or_msgs, w_stack, vas, bsum):
    n = N_NODES
    d = w_stack.shape[1]
    grid = n // PROJ_BLK
    full = lambda shp: pl.BlockSpec(shp, lambda i: (0,) * len(shp))
    row = lambda c: pl.BlockSpec((PROJ_BLK, c), lambda i: (i, 0))
    if bsum is None:
        body = _proj_body_l0
        ins = list(xs_or_msgs) + [w_stack] + list(vas)
        in_specs = [row(d)] * 4 + [full((8, d, H * C))] + [full((d, 16))] * 4
    else:
        body = _proj_body_l
        ins = list(xs_or_msgs) + [bsum, w_stack] + list(vas)
        in_specs = ([row(C)] * 4 + [full((4, C)), full((8, d, H * C))]
                    + [full((d, 16))] * 4)
    out_specs = [row(H * C)] * 8 + [row(16)] * 4
    out_shape = ([jax.ShapeDtypeStruct((n, H * C), jnp.float32)] * 8
                 + [jax.ShapeDtypeStruct((n, 16), jnp.float32)] * 4)
    res = pl.pallas_call(
        body, grid=(grid,), in_specs=in_specs, out_specs=out_specs,
        out_shape=out_shape,
    )(*ins)
    return res[:8], res[8:]


def _iota16():
    return lax.iota(jnp.int32, 16)


def _full16(v):
    return jnp.full((16,), v, jnp.int32)


def _edge_body(last, *refs):
    xs_refs = refs[0:8]
    al_refs = refs[8:12]
    src_refs = refs[12:20]
    dst_refs = refs[20:28]
    z32h, z2h = refs[28:30]
    n_out = 1 if last else 4
    out_refs = refs[30:30 + n_out]
    (acc, den, srcc, dstc, dstadj, alsv, aldv, exall, exsrc, denv, xsv,
     msgv) = refs[30 + n_out:]

    cid = lax.axis_index("c")
    tid = lax.axis_index("s")
    rbase = tid * RH_A
    ebase = tid * E_TILE
    iota = _iota16()

    def _zx(k, carry):
        f = iota + 16 * k
        plsc.store_scatter(exsrc, [lax.shift_right_logical(f, 3),
                                   lax.bitwise_and(f, 7)],
                           jnp.zeros((16,), jnp.float32))
        return carry
    lax.fori_loop(0, CH * 8 // 16, _zx, 0)

    def split_copy(src_fn, dst_fn):
        @pl.when(tid < 15)
        def _():
            pltpu.sync_copy(src_fn(rbase, RH_A), dst_fn(rbase, RH_A))

        @pl.when(tid == 15)
        def _():
            pltpu.sync_copy(src_fn(rbase, RH_LAST), dst_fn(rbase, RH_LAST))

    def zero_acc():
        split_copy(lambda o, s: z32h.at[pl.ds(o, s), :],
                   lambda o, s: acc.at[pl.ds(o, s), :])

    def zero_den():
        split_copy(lambda o, s: z2h.at[pl.ds(o, s), :],
                   lambda o, s: den.at[pl.ds(o, s), :])

    def clamp_dst(h0):
        def grp(g, carry):
            e16 = iota + 16 * g
            d = plsc.load_gather(dstc, [e16])
            cl = jnp.minimum(jnp.maximum(d - h0, 0), NHALF - 1)
            dstadj[pl.ds(g * 16, 16)] = cl
            return carry
        lax.fori_loop(0, CH // 16, grp, 0)

    def pass_a(r, h0):
        al_s = al_refs[REL_SRC[r]]
        al_d = al_refs[REL_DST[r]]
        scol = SRC_COL[r]
        dcol = DST_COL[r]

        def chunk(i, carry):
            eb = ebase + i * CH
            pltpu.sync_copy(src_refs[r].at[pl.ds(eb, CH)], srcc)
            pltpu.sync_copy(dst_refs[r].at[pl.ds(eb, CH)], dstc)
            pltpu.sync_copy(al_s.at[srcc], alsv)
            pltpu.sync_copy(al_d.at[dstc], aldv)
            clamp_dst(h0)
            exc = exall.at[pl.ds(i * CH, CH), :]

            def pair(j, carry2):
                f = iota + 16 * j
                row = lax.shift_right_logical(f, 1)
                col = lax.bitwise_and(f, 1)
                a = (plsc.load_gather(alsv, [row, scol + col])
                     + plsc.load_gather(aldv, [row, dcol + col]))
                a = jnp.where(a >= 0.0, a, 0.2 * a)
                e = jnp.exp(a)
                plsc.store_scatter(exc, [row, col], e)
                d = plsc.load_gather(dstc, [row])
                inh = jnp.logical_and(d >= h0, d < h0 + NHALF)
                em = jnp.where(inh, e, 0.0)
                plsc.store_scatter(exsrc, [row, col], em)
                return carry2

            lax.fori_loop(0, CH * 2 // 16, pair, 0)
            pltpu.sync_copy(exsrc, den.at[dstadj], add=True)
            return carry

        lax.fori_loop(0, NCH, chunk, 0)

    def pass_b(r, h0):
        xs_r = xs_refs[r]

        def chunk(i, carry):
            eb = ebase + i * CH
            pltpu.sync_copy(src_refs[r].at[pl.ds(eb, CH)], srcc)
            pltpu.sync_copy(dst_refs[r].at[pl.ds(eb, CH)], dstc)
            clamp_dst(h0)
            pltpu.sync_copy(den.at[dstadj], denv)
            pltpu.sync_copy(xs_r.at[srcc], xsv)
            exc = exall.at[pl.ds(i * CH, CH), :]

            def grp(g, carry2):
                e16 = iota + 16 * g
                d = plsc.load_gather(dstc, [e16])
                inh = jnp.logical_and(d >= h0, d < h0 + NHALF)
                sel = jnp.where(inh, 0.5, 0.0)
                a0 = (plsc.load_gather(exc, [e16, _full16(0)])
                      / (plsc.load_gather(denv, [e16, _full16(0)]) + 1e-16))
                a1 = (plsc.load_gather(exc, [e16, _full16(1)])
                      / (plsc.load_gather(denv, [e16, _full16(1)]) + 1e-16))
                a0 = a0 * sel
                a1 = a1 * sel

                def feat(j, carry3):
                    jf = jnp.full((16,), j, jnp.int32)
                    m = (plsc.load_gather(xsv, [e16, jf]) * a0
                         + plsc.load_gather(xsv, [e16, jf + C]) * a1)
                    plsc.store_scatter(msgv, [e16, jf], m)
                    return carry3

                lax.fori_loop(0, C, feat, 0)
                return carry2

            lax.fori_loop(0, CH // 16, grp, 0)
            pltpu.sync_copy(msgv, acc.at[dstadj], add=True)
            return carry

        lax.fori_loop(0, NCH, chunk, 0)

    def run_half(out_ref, rels, h0):
        zero_acc()
        for r in rels:
            zero_den()
            plsc.subcore_barrier()
            pass_a(r, h0)
            plsc.subcore_barrier()
            pass_b(r, h0)
            plsc.subcore_barrier()
        split_copy(lambda o, s: acc.at[pl.ds(o, s), :],
                   lambda o, s: out_ref.at[pl.ds(h0 + o, s), :])
        plsc.subcore_barrier()

    def run_plan(plan):
        for t, rels in plan:
            for half in range(2):
                run_half(out_refs[t], rels, half * NHALF)

    if last:
        @pl.when(cid == 0)
        def _():
            run_half(out_refs[0], [2, 4, 6], 0)

        @pl.when(cid == 1)
        def _():
            run_half(out_refs[0], [2, 4, 6], NHALF)
    else:
        @pl.when(cid == 0)
        def _():
            run_plan(CORE_PLAN[0])

        @pl.when(cid == 1)
        def _():
            run_plan(CORE_PLAN[1])


def _edge_phase(xs_list, al_list, srcs, dsts, z32h, z2h, last=False):
    mesh = plsc.VectorSubcoreMesh(core_axis_name="c", subcore_axis_name="s")
    n_out = 1 if last else 4
    f = pl.kernel(
        functools.partial(_edge_body, last),
        out_type=[jax.ShapeDtypeStruct((N_NODES, C), jnp.float32)] * n_out,
        mesh=mesh,
        compiler_params=pltpu.CompilerParams(needs_layout_passes=False,
                                             use_tc_tiling_on_sc=False),
        scratch_types=[
            pltpu.VMEM_SHARED((NHALF, C), jnp.float32),
            pltpu.VMEM_SHARED((NHALF, 8), jnp.float32),
            pltpu.VMEM((CH,), jnp.int32),
            pltpu.VMEM((CH,), jnp.int32),
            pltpu.VMEM((CH,), jnp.int32),
            pltpu.VMEM((CH, 16), jnp.float32),
            pltpu.VMEM((CH, 16), jnp.float32),
            pltpu.VMEM((E_TILE, 2), jnp.float32),
            pltpu.VMEM((CH, 8), jnp.float32),
            pltpu.VMEM((CH, 8), jnp.float32),
            pltpu.VMEM((CH, H * C), jnp.float32),
            pltpu.VMEM((CH, C), jnp.float32),
        ],
    )
    return f(*xs_list, *al_list, *srcs, *dsts, z32h, z2h)


def _mlp_body(m_ref, bsum, wz0, bz0, wz1, bz1, wz2, bz2, wv0, bv0, wv1, bv1,
              wv2, bv2, pz_ref, val_ref):
    x = jnp.maximum(m_ref[...] + bsum[...], 0.0)
    h = jnp.maximum(jnp.dot(x, wz0[...], preferred_element_type=jnp.float32) + bz0[...], 0.0)
    h = jnp.maximum(jnp.dot(h, wz1[...], preferred_element_type=jnp.float32) + bz1[...], 0.0)
    pz_ref[...] = jax.nn.sigmoid(
        jnp.dot(h, wz2[...], preferred_element_type=jnp.float32) + bz2[...])
    h = jnp.maximum(jnp.dot(x, wv0[...], preferred_element_type=jnp.float32) + bv0[...], 0.0)
    h = jnp.maximum(jnp.dot(h, wv1[...], preferred_element_type=jnp.float32) + bv1[...], 0.0)
    val_ref[...] = jnp.dot(h, wv2[...], preferred_element_type=jnp.float32) + bv2[...]


def _mlp_heads(msg_gene, bsum_gene, Wz0, bz0, Wz1, bz1, Wz2, bz2,
               Wv0, bv0, Wv1, bv1, Wv2, bv2):
    n = msg_gene.shape[0]
    grid = n // MLP_BLK
    full = lambda shp: pl.BlockSpec(shp, lambda i: (0,) * len(shp))
    return pl.pallas_call(
        _mlp_body,
        grid=(grid,),
        in_specs=[
            pl.BlockSpec((MLP_BLK, C), lambda i: (i, 0)),
            full((1, C)),
            full((C, D_INNER)), full((1, D_INNER)),
            full((D_INNER, D_INNER)), full((1, D_INNER)),
            full((D_INNER, 1)), full((1, 1)),
            full((C, D_INNER)), full((1, D_INNER)),
            full((D_INNER, D_INNER)), full((1, D_INNER)),
            full((D_INNER, 1)), full((1, 1)),
        ],
        out_specs=[
            pl.BlockSpec((MLP_BLK, 1), lambda i: (i, 0)),
            pl.BlockSpec((MLP_BLK, 1), lambda i: (i, 0)),
        ],
        out_shape=[
            jax.ShapeDtypeStruct((n, 1), jnp.float32),
            jax.ShapeDtypeStruct((n, 1), jnp.float32),
        ],
    )(msg_gene, bsum_gene.reshape(1, -1),
      Wz0, bz0.reshape(1, -1), Wz1, bz1.reshape(1, -1), Wz2, bz2.reshape(1, -1),
      Wv0, bv0.reshape(1, -1), Wv1, bv1.reshape(1, -1), Wv2, bv2.reshape(1, -1))


def _fold_logit_vectors(W, att):
    d = W.shape[1]
    return jnp.einsum("rdhc,rhc->rdh", W.reshape(8, d, H, C), att)


def _build_al_weights(W_src_l, W_dst_l, att_src_l, att_dst_l):
    vs = _fold_logit_vectors(W_src_l, att_src_l)
    vd = _fold_logit_vectors(W_dst_l, att_dst_l)
    d = W_src_l.shape[1]
    vas = []
    for t in range(4):
        cols = []
        for r in range(8):
            if REL_SRC[r] == t:
                cols.append(vs[r])
        for r in range(8):
            if REL_DST[r] == t:
                cols.append(vd[r])
        va = jnp.concatenate(cols, axis=1)
        va = jnp.pad(va, ((0, 0), (0, 16 - va.shape[1])))
        vas.append(va)
    return vas


def _bias_sums(b_l):
    return jnp.stack(
        [sum(b_l[r] for r in range(8) if REL_DST[r] == t) for t in range(4)])


def kernel(x_tad, x_atac, x_gene, x_protein, ei0, ei1, ei2, ei3, ei4, ei5, ei6,
           ei7, W0_src, W0_dst, att0_src, att0_dst, b0, W_src, W_dst, att_src,
           att_dst, b, Wz0, bz0, Wz1, bz1, Wz2, bz2, Wv0, bv0, Wv1, bv1, Wv2, bv2):
    xs = [x_tad, x_atac, x_gene, x_protein]
    eis = [ei0, ei1, ei2, ei3, ei4, ei5, ei6, ei7]
    srcs = [e[0] for e in eis]
    dsts = [e[1] for e in eis]
    z32h = jnp.zeros((NHALF, C), jnp.float32)
    z2h = jnp.zeros((NHALF, 8), jnp.float32)

    msgs = None
    for l in range(4):
        if l == 0:
            Ws_l, Wd_l, as_l, ad_l = W0_src, W0_dst, att0_src, att0_dst
            bsum = None
        else:
            Ws_l, Wd_l, as_l, ad_l = (W_src[l - 1], W_dst[l - 1],
                                      att_src[l - 1], att_dst[l - 1])
            bsum = _bias_sums(b[l - 1])
        vas = _build_al_weights(Ws_l, Wd_l, as_l, ad_l)
        src_in = xs if l == 0 else list(msgs)
        xs_proj, al_list = _proj_call(src_in, Ws_l, vas, bsum)
        msgs = _edge_phase(xs_proj, al_list, srcs, dsts, z32h, z2h,
                           last=(l == 3))

    bsum_last = _bias_sums(b[2])
    p_zero, values = _mlp_heads(msgs[0], bsum_last[2], Wz0, bz0, Wz1, bz1,
                                Wz2, bz2, Wv0, bv0, Wv1, bv1, Wv2, bv2)
    zeros = jax.random.bernoulli(jax.random.key(1), p_zero).astype(jnp.float32)
    return (p_zero, zeros, values)


# speedup vs baseline: 2.5548x; 1.4933x over previous
import functools

import jax
import jax.numpy as jnp
from jax import lax
from jax.experimental import pallas as pl
from jax.experimental.pallas import tpu as pltpu
from jax.experimental.pallas import tpu_sc as plsc

N_NODES = 50000
E = 64000
D_IN = 128
H = 2
C = 32
D_INNER = 512
REL_SRC = [0, 1, 0, 2, 1, 2, 3, 2]
REL_DST = [1, 0, 2, 0, 2, 1, 2, 3]

N_TILES = 16
RPT_A = 3128
RPT_LAST = N_NODES - 15 * RPT_A
ND = 2 * N_NODES
RPTD_A = 6256
RPTD_LAST = ND - 15 * RPTD_A
E_TILE = E // N_TILES
CH = 80
NCH = E_TILE // CH

SRC_COL = {}
DST_COL = {}
for _t in range(4):
    _c = 0
    for _r in range(8):
        if REL_SRC[_r] == _t:
            SRC_COL[_r] = _c
            _c += 2
    for _r in range(8):
        if REL_DST[_r] == _t:
            DST_COL[_r] = _c
            _c += 2

CORE_PLAN = {
    0: [(0, [1, 3]), (1, [0, 5])],
    1: [(2, [2, 4, 6]), (3, [7])],
}
LAST_PLAN = {0: [2], 1: [4, 6]}

MLP_BLK = 2000
PROJ_BLK = 2000


def _proj_body_l0(x0, x1, x2, x3, w, va0, va1, va2, va3, *outs):
    xs = (x0[...], x1[...], x2[...], x3[...])
    vas = (va0, va1, va2, va3)
    for r in range(8):
        outs[r][...] = jnp.dot(xs[REL_SRC[r]], w[r], preferred_element_type=jnp.float32)
    for t in range(4):
        outs[8 + t][...] = jnp.dot(xs[t], vas[t][...], preferred_element_type=jnp.float32)


def _proj_body_l(m0, m1, m2, m3, bsum, w, va0, va1, va2, va3, *outs):
    ms = (m0[...], m1[...], m2[...], m3[...])
    bs = bsum[...]
    vas = (va0, va1, va2, va3)
    xs = [jnp.maximum(ms[t] + bs[t], 0.0) for t in range(4)]
    for r in range(8):
        outs[r][...] = jnp.dot(xs[REL_SRC[r]], w[r], preferred_element_type=jnp.float32)
    for t in range(4):
        outs[8 + t][...] = jnp.dot(xs[t], vas[t][...], preferred_element_type=jnp.float32)


def _proj_call(xs_or_msgs, w_stack, vas, bsum):
    n = N_NODES
    d = w_stack.shape[1]
    grid = n // PROJ_BLK
    full = lambda shp: pl.BlockSpec(shp, lambda i: (0,) * len(shp))
    row = lambda c: pl.BlockSpec((PROJ_BLK, c), lambda i: (i, 0))
    if bsum is None:
        body = _proj_body_l0
        ins = list(xs_or_msgs) + [w_stack] + list(vas)
        in_specs = [row(d)] * 4 + [full((8, d, H * C))] + [full((d, 16))] * 4
    else:
        body = _proj_body_l
        ins = list(xs_or_msgs) + [bsum, w_stack] + list(vas)
        in_specs = ([row(C)] * 4 + [full((4, C)), full((8, d, H * C))]
                    + [full((d, 16))] * 4)
    out_specs = [row(H * C)] * 8 + [row(16)] * 4
    out_shape = ([jax.ShapeDtypeStruct((n, H * C), jnp.float32)] * 8
                 + [jax.ShapeDtypeStruct((n, 16), jnp.float32)] * 4)
    res = pl.pallas_call(
        body, grid=(grid,), in_specs=in_specs, out_specs=out_specs,
        out_shape=out_shape,
    )(*ins)
    return res[:8], res[8:]


def _iota16():
    return lax.iota(jnp.int32, 16)


def _full16(v):
    return jnp.full((16,), v, jnp.int32)


def _edge_body(last, *refs):
    xs_refs = refs[0:8]
    al_refs = refs[8:12]
    src_refs = refs[12:20]
    dst_refs = refs[20:28]
    z32h, zdh = refs[28:30]
    n_out = 2 if last else 4
    out_refs = refs[30:30 + n_out]
    (acc, den, srcc, dstc, idx0, idx1, ex0, ex1, den0v, den1v, xsv,
     alsv, aldv, msgv) = refs[30 + n_out:]

    cid = lax.axis_index("c")
    tid = lax.axis_index("s")
    ebase = tid * E_TILE
    iota = _iota16()

    def split_rows(src_fn, dst_fn):
        @pl.when(tid < 15)
        def _():
            pltpu.sync_copy(src_fn(tid * RPT_A, RPT_A),
                            dst_fn(tid * RPT_A, RPT_A))

        @pl.when(tid == 15)
        def _():
            pltpu.sync_copy(src_fn(tid * RPT_A, RPT_LAST),
                            dst_fn(tid * RPT_A, RPT_LAST))

    def zero_acc():
        split_rows(lambda o, s: z32h.at[pl.ds(o, s), :],
                   lambda o, s: acc.at[pl.ds(o, s), :])

    def zero_den():
        @pl.when(tid < 15)
        def _():
            pltpu.sync_copy(zdh.at[pl.ds(tid * RPTD_A, RPTD_A)],
                            den.at[pl.ds(tid * RPTD_A, RPTD_A)])

        @pl.when(tid == 15)
        def _():
            pltpu.sync_copy(zdh.at[pl.ds(tid * RPTD_A, RPTD_LAST)],
                            den.at[pl.ds(tid * RPTD_A, RPTD_LAST)])

    def compute_ex(r):
        scol = SRC_COL[r]
        dcol = DST_COL[r]

        def grp(g, carry):
            e16 = iota + 16 * g
            d = plsc.load_gather(dstc, [e16])
            idx0[pl.ds(g * 16, 16)] = 2 * d
            idx1[pl.ds(g * 16, 16)] = 2 * d + 1
            for h, exb in ((0, ex0), (1, ex1)):
                a = (plsc.load_gather(alsv, [e16, _full16(scol + h)])
                     + plsc.load_gather(aldv, [e16, _full16(dcol + h)]))
                a = jnp.where(a >= 0.0, a, 0.2 * a)
                exb[pl.ds(g * 16, 16)] = jnp.exp(a)
            return carry

        lax.fori_loop(0, CH // 16, grp, 0)

    def pass_a(r):
        al_s = al_refs[REL_SRC[r]]
        al_d = al_refs[REL_DST[r]]

        def chunk(i, carry):
            eb = ebase + i * CH
            pltpu.sync_copy(src_refs[r].at[pl.ds(eb, CH)], srcc)
            pltpu.sync_copy(dst_refs[r].at[pl.ds(eb, CH)], dstc)
            pltpu.sync_copy(al_s.at[srcc], alsv)
            pltpu.sync_copy(al_d.at[dstc], aldv)
            compute_ex(r)
            pltpu.sync_copy(ex0, den.at[idx0], add=True)
            pltpu.sync_copy(ex1, den.at[idx1], add=True)
            return carry

        lax.fori_loop(0, NCH, chunk, 0)

    def pass_b(r):
        al_s = al_refs[REL_SRC[r]]
        al_d = al_refs[REL_DST[r]]
        xs_r = xs_refs[r]

        def chunk(i, carry):
            eb = ebase + i * CH
            pltpu.sync_copy(src_refs[r].at[pl.ds(eb, CH)], srcc)
            pltpu.sync_copy(dst_refs[r].at[pl.ds(eb, CH)], dstc)
            pltpu.sync_copy(al_s.at[srcc], alsv)
            pltpu.sync_copy(al_d.at[dstc], aldv)
            compute_ex(r)
            pltpu.sync_copy(den.at[idx0], den0v)
            pltpu.sync_copy(den.at[idx1], den1v)
            pltpu.sync_copy(xs_r.at[srcc], xsv)

            def grp(g, carry2):
                e16 = iota + 16 * g
                a0 = (plsc.load_gather(ex0, [e16]) * 0.5
                      / (plsc.load_gather(den0v, [e16]) + 1e-16))
                a1 = (plsc.load_gather(ex1, [e16]) * 0.5
                      / (plsc.load_gather(den1v, [e16]) + 1e-16))

                def feat(j, carry3):
                    jf = jnp.full((16,), j, jnp.int32)
                    m = (plsc.load_gather(xsv, [e16, jf]) * a0
                         + plsc.load_gather(xsv, [e16, jf + C]) * a1)
                    plsc.store_scatter(msgv, [e16, jf], m)
                    return carry3

                lax.fori_loop(0, C, feat, 0)
                return carry2

            lax.fori_loop(0, CH // 16, grp, 0)
            pltpu.sync_copy(msgv, acc.at[dstc], add=True)
            return carry

        lax.fori_loop(0, NCH, chunk, 0)

    def run_group(out_ref, rels):
        zero_acc()
        for r in rels:
            zero_den()
            plsc.subcore_barrier()
            pass_a(r)
            plsc.subcore_barrier()
            pass_b(r)
            plsc.subcore_barrier()
        split_rows(lambda o, s: acc.at[pl.ds(o, s), :],
                   lambda o, s: out_ref.at[pl.ds(o, s), :])
        plsc.subcore_barrier()

    if last:
        @pl.when(cid == 0)
        def _():
            run_group(out_refs[0], LAST_PLAN[0])

        @pl.when(cid == 1)
        def _():
            run_group(out_refs[1], LAST_PLAN[1])
    else:
        @pl.when(cid == 0)
        def _():
            for t, rels in CORE_PLAN[0]:
                run_group(out_refs[t], rels)

        @pl.when(cid == 1)
        def _():
            for t, rels in CORE_PLAN[1]:
                run_group(out_refs[t], rels)


def _edge_phase(xs_list, al_list, srcs, dsts, z32h, zdh, last=False):
    mesh = plsc.VectorSubcoreMesh(core_axis_name="c", subcore_axis_name="s")
    n_out = 2 if last else 4
    f = pl.kernel(
        functools.partial(_edge_body, last),
        out_type=[jax.ShapeDtypeStruct((N_NODES, C), jnp.float32)] * n_out,
        mesh=mesh,
        compiler_params=pltpu.CompilerParams(needs_layout_passes=False,
                                             use_tc_tiling_on_sc=False),
        scratch_types=[
            pltpu.VMEM_SHARED((N_NODES, C), jnp.float32),
            pltpu.VMEM_SHARED((ND,), jnp.float32),
            pltpu.VMEM((CH,), jnp.int32),
            pltpu.VMEM((CH,), jnp.int32),
            pltpu.VMEM((CH,), jnp.int32),
            pltpu.VMEM((CH,), jnp.int32),
            pltpu.VMEM((CH,), jnp.float32),
            pltpu.VMEM((CH,), jnp.float32),
            pltpu.VMEM((CH,), jnp.float32),
            pltpu.VMEM((CH,), jnp.float32),
            pltpu.VMEM((CH, H * C), jnp.float32),
            pltpu.VMEM((CH, 16), jnp.float32),
            pltpu.VMEM((CH, 16), jnp.float32),
            pltpu.VMEM((CH, C), jnp.float32),
        ],
    )
    return f(*xs_list, *al_list, *srcs, *dsts, z32h, zdh)


def _mlp_body(m_ref, m2_ref, bsum, wz0, bz0, wz1, bz1, wz2, bz2, wv0, bv0,
              wv1, bv1, wv2, bv2, pz_ref, val_ref):
    x = jnp.maximum(m_ref[...] + m2_ref[...] + bsum[...], 0.0)
    h = jnp.maximum(jnp.dot(x, wz0[...], preferred_element_type=jnp.float32) + bz0[...], 0.0)
    h = jnp.maximum(jnp.dot(h, wz1[...], preferred_element_type=jnp.float32) + bz1[...], 0.0)
    pz_ref[...] = jax.nn.sigmoid(
        jnp.dot(h, wz2[...], preferred_element_type=jnp.float32) + bz2[...])
    h = jnp.maximum(jnp.dot(x, wv0[...], preferred_element_type=jnp.float32) + bv0[...], 0.0)
    h = jnp.maximum(jnp.dot(h, wv1[...], preferred_element_type=jnp.float32) + bv1[...], 0.0)
    val_ref[...] = jnp.dot(h, wv2[...], preferred_element_type=jnp.float32) + bv2[...]


def _mlp_heads(msg_gene, msg_gene2, bsum_gene, Wz0, bz0, Wz1, bz1, Wz2, bz2,
               Wv0, bv0, Wv1, bv1, Wv2, bv2):
    n = msg_gene.shape[0]
    grid = n // MLP_BLK
    full = lambda shp: pl.BlockSpec(shp, lambda i: (0,) * len(shp))
    return pl.pallas_call(
        _mlp_body,
        grid=(grid,),
        in_specs=[
            pl.BlockSpec((MLP_BLK, C), lambda i: (i, 0)),
            pl.BlockSpec((MLP_BLK, C), lambda i: (i, 0)),
            full((1, C)),
            full((C, D_INNER)), full((1, D_INNER)),
            full((D_INNER, D_INNER)), full((1, D_INNER)),
            full((D_INNER, 1)), full((1, 1)),
            full((C, D_INNER)), full((1, D_INNER)),
            full((D_INNER, D_INNER)), full((1, D_INNER)),
            full((D_INNER, 1)), full((1, 1)),
        ],
        out_specs=[
            pl.BlockSpec((MLP_BLK, 1), lambda i: (i, 0)),
            pl.BlockSpec((MLP_BLK, 1), lambda i: (i, 0)),
        ],
        out_shape=[
            jax.ShapeDtypeStruct((n, 1), jnp.float32),
            jax.ShapeDtypeStruct((n, 1), jnp.float32),
        ],
    )(msg_gene, msg_gene2, bsum_gene.reshape(1, -1),
      Wz0, bz0.reshape(1, -1), Wz1, bz1.reshape(1, -1), Wz2, bz2.reshape(1, -1),
      Wv0, bv0.reshape(1, -1), Wv1, bv1.reshape(1, -1), Wv2, bv2.reshape(1, -1))


def _fold_logit_vectors(W, att):
    d = W.shape[1]
    return jnp.einsum("rdhc,rhc->rdh", W.reshape(8, d, H, C), att)


def _build_al_weights(W_src_l, W_dst_l, att_src_l, att_dst_l):
    vs = _fold_logit_vectors(W_src_l, att_src_l)
    vd = _fold_logit_vectors(W_dst_l, att_dst_l)
    d = W_src_l.shape[1]
    vas = []
    for t in range(4):
        cols = []
        for r in range(8):
            if REL_SRC[r] == t:
                cols.append(vs[r])
        for r in range(8):
            if REL_DST[r] == t:
                cols.append(vd[r])
        va = jnp.concatenate(cols, axis=1)
        va = jnp.pad(va, ((0, 0), (0, 16 - va.shape[1])))
        vas.append(va)
    return vas


def _bias_sums(b_l):
    return jnp.stack(
        [sum(b_l[r] for r in range(8) if REL_DST[r] == t) for t in range(4)])


def kernel(x_tad, x_atac, x_gene, x_protein, ei0, ei1, ei2, ei3, ei4, ei5, ei6,
           ei7, W0_src, W0_dst, att0_src, att0_dst, b0, W_src, W_dst, att_src,
           att_dst, b, Wz0, bz0, Wz1, bz1, Wz2, bz2, Wv0, bv0, Wv1, bv1, Wv2, bv2):
    xs = [x_tad, x_atac, x_gene, x_protein]
    eis = [ei0, ei1, ei2, ei3, ei4, ei5, ei6, ei7]
    srcs = [e[0] for e in eis]
    dsts = [e[1] for e in eis]
    z32h = jnp.zeros((N_NODES, C), jnp.float32)
    zdh = jnp.zeros((ND,), jnp.float32)

    msgs = None
    for l in range(4):
        if l == 0:
            Ws_l, Wd_l, as_l, ad_l = W0_src, W0_dst, att0_src, att0_dst
            bsum = None
        else:
            Ws_l, Wd_l, as_l, ad_l = (W_src[l - 1], W_dst[l - 1],
                                      att_src[l - 1], att_dst[l - 1])
            bsum = _bias_sums(b[l - 1])
        vas = _build_al_weights(Ws_l, Wd_l, as_l, ad_l)
        src_in = xs if l == 0 else list(msgs)
        xs_proj, al_list = _proj_call(src_in, Ws_l, vas, bsum)
        msgs = _edge_phase(xs_proj, al_list, srcs, dsts, z32h, zdh,
                           last=(l == 3))

    bsum_last = _bias_sums(b[2])
    p_zero, values = _mlp_heads(msgs[0], msgs[1], bsum_last[2], Wz0, bz0, Wz1, bz1,
                                Wz2, bz2, Wv0, bv0, Wv1, bv1, Wv2, bv2)
    zeros = jax.random.bernoulli(jax.random.key(1), p_zero).astype(jnp.float32)
    return (p_zero, zeros, values)


# speedup vs baseline: 2.8708x; 1.1237x over previous
import functools

import jax
import jax.numpy as jnp
from jax import lax
from jax.experimental import pallas as pl
from jax.experimental.pallas import tpu as pltpu
from jax.experimental.pallas import tpu_sc as plsc

N_NODES = 50000
E = 64000
D_IN = 128
H = 2
C = 32
D_INNER = 512
REL_SRC = [0, 1, 0, 2, 1, 2, 3, 2]
REL_DST = [1, 0, 2, 0, 2, 1, 2, 3]

N_TILES = 16
RPT_A = 3128
RPT_LAST = N_NODES - 15 * RPT_A
ND = 2 * N_NODES
RPTD_A = 6256
RPTD_LAST = ND - 15 * RPTD_A
E_TILE = E // N_TILES
CH = 80
NCH = E_TILE // CH

SRC_COL = {}
DST_COL = {}
for _t in range(4):
    _c = 0
    for _r in range(8):
        if REL_SRC[_r] == _t:
            SRC_COL[_r] = _c
            _c += 2
    for _r in range(8):
        if REL_DST[_r] == _t:
            DST_COL[_r] = _c
            _c += 2

CORE_PLAN = {
    0: [(0, [1, 3]), (1, [0, 5])],
    1: [(2, [2, 4, 6]), (3, [7])],
}
LAST_PLAN = {0: [2], 1: [4, 6]}

MLP_BLK = 2000
PROJ_BLK = 2000


def _proj_body_l0(x0, x1, x2, x3, w, va0, va1, va2, va3, *outs):
    xs = (x0[...], x1[...], x2[...], x3[...])
    vas = (va0, va1, va2, va3)
    for r in range(8):
        outs[r][...] = jnp.dot(xs[REL_SRC[r]], w[r], preferred_element_type=jnp.float32)
    for t in range(4):
        outs[8 + t][...] = jnp.dot(xs[t], vas[t][...], preferred_element_type=jnp.float32)


def _proj_body_l(m0, m1, m2, m3, bsum, w, va0, va1, va2, va3, *outs):
    ms = (m0[...], m1[...], m2[...], m3[...])
    bs = bsum[...]
    vas = (va0, va1, va2, va3)
    xs = [jnp.maximum(ms[t] + bs[t], 0.0) for t in range(4)]
    for r in range(8):
        outs[r][...] = jnp.dot(xs[REL_SRC[r]], w[r], preferred_element_type=jnp.float32)
    for t in range(4):
        outs[8 + t][...] = jnp.dot(xs[t], vas[t][...], preferred_element_type=jnp.float32)


def _proj_call(xs_or_msgs, w_stack, vas, bsum):
    n = N_NODES
    d = w_stack.shape[1]
    grid = n // PROJ_BLK
    full = lambda shp: pl.BlockSpec(shp, lambda i: (0,) * len(shp))
    row = lambda c: pl.BlockSpec((PROJ_BLK, c), lambda i: (i, 0))
    if bsum is None:
        body = _proj_body_l0
        ins = list(xs_or_msgs) + [w_stack] + list(vas)
        in_specs = [row(d)] * 4 + [full((8, d, H * C))] + [full((d, 16))] * 4
    else:
        body = _proj_body_l
        ins = list(xs_or_msgs) + [bsum, w_stack] + list(vas)
        in_specs = ([row(C)] * 4 + [full((4, C)), full((8, d, H * C))]
                    + [full((d, 16))] * 4)
    out_specs = [row(H * C)] * 8 + [row(16)] * 4
    out_shape = ([jax.ShapeDtypeStruct((n, H * C), jnp.float32)] * 8
                 + [jax.ShapeDtypeStruct((n, 16), jnp.float32)] * 4)
    res = pl.pallas_call(
        body, grid=(grid,), in_specs=in_specs, out_specs=out_specs,
        out_shape=out_shape,
    )(*ins)
    return res[:8], res[8:]


def _iota16():
    return lax.iota(jnp.int32, 16)


def _full16(v):
    return jnp.full((16,), v, jnp.int32)


def _edge_body(last, *refs):
    xs_refs = refs[0:8]
    al_refs = refs[8:12]
    src_refs = refs[12:20]
    dst_refs = refs[20:28]
    z32h, zdh = refs[28:30]
    n_out = 2 if last else 4
    out_refs = refs[30:30 + n_out]
    (acc, den, srcc, dstc, idx0, idx1, exa0, exa1, den0v, den1v, xsv,
     alsv, aldv, msgv) = refs[30 + n_out:]

    cid = lax.axis_index("c")
    tid = lax.axis_index("s")
    ebase = tid * E_TILE
    iota = _iota16()

    def split_rows(src_fn, dst_fn):
        @pl.when(tid < 15)
        def _():
            pltpu.sync_copy(src_fn(tid * RPT_A, RPT_A),
                            dst_fn(tid * RPT_A, RPT_A))

        @pl.when(tid == 15)
        def _():
            pltpu.sync_copy(src_fn(tid * RPT_A, RPT_LAST),
                            dst_fn(tid * RPT_A, RPT_LAST))

    def zero_acc():
        split_rows(lambda o, s: z32h.at[pl.ds(o, s), :],
                   lambda o, s: acc.at[pl.ds(o, s), :])

    def zero_den():
        @pl.when(tid < 15)
        def _():
            pltpu.sync_copy(zdh.at[pl.ds(tid * RPTD_A, RPTD_A)],
                            den.at[pl.ds(tid * RPTD_A, RPTD_A)])

        @pl.when(tid == 15)
        def _():
            pltpu.sync_copy(zdh.at[pl.ds(tid * RPTD_A, RPTD_LAST)],
                            den.at[pl.ds(tid * RPTD_A, RPTD_LAST)])

    def build_idx():
        def grp(g, carry):
            e16 = iota + 16 * g
            d = plsc.load_gather(dstc, [e16])
            idx0[pl.ds(g * 16, 16)] = 2 * d
            idx1[pl.ds(g * 16, 16)] = 2 * d + 1
            return carry

        lax.fori_loop(0, CH // 16, grp, 0)

    def compute_ex(r, i):
        scol = SRC_COL[r]
        dcol = DST_COL[r]

        def grp(g, carry):
            e16 = iota + 16 * g
            d = plsc.load_gather(dstc, [e16])
            idx0[pl.ds(g * 16, 16)] = 2 * d
            idx1[pl.ds(g * 16, 16)] = 2 * d + 1
            for h, exb in ((0, exa0), (1, exa1)):
                a = (plsc.load_gather(alsv, [e16, _full16(scol + h)])
                     + plsc.load_gather(aldv, [e16, _full16(dcol + h)]))
                a = jnp.where(a >= 0.0, a, 0.2 * a)
                exb[pl.ds(i * CH + g * 16, 16)] = jnp.exp(a)
            return carry

        lax.fori_loop(0, CH // 16, grp, 0)

    def pass_a(r):
        al_s = al_refs[REL_SRC[r]]
        al_d = al_refs[REL_DST[r]]

        def chunk(i, carry):
            eb = ebase + i * CH
            pltpu.sync_copy(src_refs[r].at[pl.ds(eb, CH)], srcc)
            pltpu.sync_copy(dst_refs[r].at[pl.ds(eb, CH)], dstc)
            pltpu.sync_copy(al_s.at[srcc], alsv)
            pltpu.sync_copy(al_d.at[dstc], aldv)
            compute_ex(r, i)
            pltpu.sync_copy(exa0.at[pl.ds(i * CH, CH)], den.at[idx0], add=True)
            pltpu.sync_copy(exa1.at[pl.ds(i * CH, CH)], den.at[idx1], add=True)
            return carry

        lax.fori_loop(0, NCH, chunk, 0)

    def pass_b(r):
        xs_r = xs_refs[r]

        def chunk(i, carry):
            eb = ebase + i * CH
            pltpu.sync_copy(src_refs[r].at[pl.ds(eb, CH)], srcc)
            pltpu.sync_copy(dst_refs[r].at[pl.ds(eb, CH)], dstc)
            build_idx()
            pltpu.sync_copy(den.at[idx0], den0v)
            pltpu.sync_copy(den.at[idx1], den1v)
            pltpu.sync_copy(xs_r.at[srcc], xsv)

            def grp(g, carry2):
                e16 = iota + 16 * g
                a0 = (plsc.load_gather(exa0, [i * CH + e16]) * 0.5
                      / (plsc.load_gather(den0v, [e16]) + 1e-16))
                a1 = (plsc.load_gather(exa1, [i * CH + e16]) * 0.5
                      / (plsc.load_gather(den1v, [e16]) + 1e-16))

                def feat(j, carry3):
                    jf = jnp.full((16,), j, jnp.int32)
                    m = (plsc.load_gather(xsv, [e16, jf]) * a0
                         + plsc.load_gather(xsv, [e16, jf + C]) * a1)
                    plsc.store_scatter(msgv, [e16, jf], m)
                    return carry3

                lax.fori_loop(0, C, feat, 0)
                return carry2

            lax.fori_loop(0, CH // 16, grp, 0)
            pltpu.sync_copy(msgv, acc.at[dstc], add=True)
            return carry

        lax.fori_loop(0, NCH, chunk, 0)

    def run_group(out_ref, rels):
        zero_acc()
        for r in rels:
            zero_den()
            plsc.subcore_barrier()
            pass_a(r)
            plsc.subcore_barrier()
            pass_b(r)
            plsc.subcore_barrier()
        split_rows(lambda o, s: acc.at[pl.ds(o, s), :],
                   lambda o, s: out_ref.at[pl.ds(o, s), :])
        plsc.subcore_barrier()

    if last:
        @pl.when(cid == 0)
        def _():
            run_group(out_refs[0], LAST_PLAN[0])

        @pl.when(cid == 1)
        def _():
            run_group(out_refs[1], LAST_PLAN[1])
    else:
        @pl.when(cid == 0)
        def _():
            for t, rels in CORE_PLAN[0]:
                run_group(out_refs[t], rels)

        @pl.when(cid == 1)
        def _():
            for t, rels in CORE_PLAN[1]:
                run_group(out_refs[t], rels)


def _edge_phase(xs_list, al_list, srcs, dsts, z32h, zdh, last=False):
    mesh = plsc.VectorSubcoreMesh(core_axis_name="c", subcore_axis_name="s")
    n_out = 2 if last else 4
    f = pl.kernel(
        functools.partial(_edge_body, last),
        out_type=[jax.ShapeDtypeStruct((N_NODES, C), jnp.float32)] * n_out,
        mesh=mesh,
        compiler_params=pltpu.CompilerParams(needs_layout_passes=False,
                                             use_tc_tiling_on_sc=False),
        scratch_types=[
            pltpu.VMEM_SHARED((N_NODES, C), jnp.float32),
            pltpu.VMEM_SHARED((ND,), jnp.float32),
            pltpu.VMEM((CH,), jnp.int32),
            pltpu.VMEM((CH,), jnp.int32),
            pltpu.VMEM((CH,), jnp.int32),
            pltpu.VMEM((CH,), jnp.int32),
            pltpu.VMEM((E_TILE,), jnp.float32),
            pltpu.VMEM((E_TILE,), jnp.float32),
            pltpu.VMEM((CH,), jnp.float32),
            pltpu.VMEM((CH,), jnp.float32),
            pltpu.VMEM((CH, H * C), jnp.float32),
            pltpu.VMEM((CH, 16), jnp.float32),
            pltpu.VMEM((CH, 16), jnp.float32),
            pltpu.VMEM((CH, C), jnp.float32),
        ],
    )
    return f(*xs_list, *al_list, *srcs, *dsts, z32h, zdh)


def _mlp_body(m_ref, m2_ref, bsum, wz0, bz0, wz1, bz1, wz2, bz2, wv0, bv0,
              wv1, bv1, wv2, bv2, pz_ref, val_ref):
    x = jnp.maximum(m_ref[...] + m2_ref[...] + bsum[...], 0.0)
    h = jnp.maximum(jnp.dot(x, wz0[...], preferred_element_type=jnp.float32) + bz0[...], 0.0)
    h = jnp.maximum(jnp.dot(h, wz1[...], preferred_element_type=jnp.float32) + bz1[...], 0.0)
    pz_ref[...] = jax.nn.sigmoid(
        jnp.dot(h, wz2[...], preferred_element_type=jnp.float32) + bz2[...])
    h = jnp.maximum(jnp.dot(x, wv0[...], preferred_element_type=jnp.float32) + bv0[...], 0.0)
    h = jnp.maximum(jnp.dot(h, wv1[...], preferred_element_type=jnp.float32) + bv1[...], 0.0)
    val_ref[...] = jnp.dot(h, wv2[...], preferred_element_type=jnp.float32) + bv2[...]


def _mlp_heads(msg_gene, msg_gene2, bsum_gene, Wz0, bz0, Wz1, bz1, Wz2, bz2,
               Wv0, bv0, Wv1, bv1, Wv2, bv2):
    n = msg_gene.shape[0]
    grid = n // MLP_BLK
    full = lambda shp: pl.BlockSpec(shp, lambda i: (0,) * len(shp))
    return pl.pallas_call(
        _mlp_body,
        grid=(grid,),
        in_specs=[
            pl.BlockSpec((MLP_BLK, C), lambda i: (i, 0)),
            pl.BlockSpec((MLP_BLK, C), lambda i: (i, 0)),
            full((1, C)),
            full((C, D_INNER)), full((1, D_INNER)),
            full((D_INNER, D_INNER)), full((1, D_INNER)),
            full((D_INNER, 1)), full((1, 1)),
            full((C, D_INNER)), full((1, D_INNER)),
            full((D_INNER, D_INNER)), full((1, D_INNER)),
            full((D_INNER, 1)), full((1, 1)),
        ],
        out_specs=[
            pl.BlockSpec((MLP_BLK, 1), lambda i: (i, 0)),
            pl.BlockSpec((MLP_BLK, 1), lambda i: (i, 0)),
        ],
        out_shape=[
            jax.ShapeDtypeStruct((n, 1), jnp.float32),
            jax.ShapeDtypeStruct((n, 1), jnp.float32),
        ],
    )(msg_gene, msg_gene2, bsum_gene.reshape(1, -1),
      Wz0, bz0.reshape(1, -1), Wz1, bz1.reshape(1, -1), Wz2, bz2.reshape(1, -1),
      Wv0, bv0.reshape(1, -1), Wv1, bv1.reshape(1, -1), Wv2, bv2.reshape(1, -1))


def _fold_logit_vectors(W, att):
    d = W.shape[1]
    return jnp.einsum("rdhc,rhc->rdh", W.reshape(8, d, H, C), att)


def _build_al_weights(W_src_l, W_dst_l, att_src_l, att_dst_l):
    vs = _fold_logit_vectors(W_src_l, att_src_l)
    vd = _fold_logit_vectors(W_dst_l, att_dst_l)
    d = W_src_l.shape[1]
    vas = []
    for t in range(4):
        cols = []
        for r in range(8):
            if REL_SRC[r] == t:
                cols.append(vs[r])
        for r in range(8):
            if REL_DST[r] == t:
                cols.append(vd[r])
        va = jnp.concatenate(cols, axis=1)
        va = jnp.pad(va, ((0, 0), (0, 16 - va.shape[1])))
        vas.append(va)
    return vas


def _bias_sums(b_l):
    return jnp.stack(
        [sum(b_l[r] for r in range(8) if REL_DST[r] == t) for t in range(4)])


def kernel(x_tad, x_atac, x_gene, x_protein, ei0, ei1, ei2, ei3, ei4, ei5, ei6,
           ei7, W0_src, W0_dst, att0_src, att0_dst, b0, W_src, W_dst, att_src,
           att_dst, b, Wz0, bz0, Wz1, bz1, Wz2, bz2, Wv0, bv0, Wv1, bv1, Wv2, bv2):
    xs = [x_tad, x_atac, x_gene, x_protein]
    eis = [ei0, ei1, ei2, ei3, ei4, ei5, ei6, ei7]
    srcs = [e[0] for e in eis]
    dsts = [e[1] for e in eis]
    z32h = jnp.zeros((N_NODES, C), jnp.float32)
    zdh = jnp.zeros((ND,), jnp.float32)

    msgs = None
    for l in range(4):
        if l == 0:
            Ws_l, Wd_l, as_l, ad_l = W0_src, W0_dst, att0_src, att0_dst
            bsum = None
        else:
            Ws_l, Wd_l, as_l, ad_l = (W_src[l - 1], W_dst[l - 1],
                                      att_src[l - 1], att_dst[l - 1])
            bsum = _bias_sums(b[l - 1])
        vas = _build_al_weights(Ws_l, Wd_l, as_l, ad_l)
        src_in = xs if l == 0 else list(msgs)
        xs_proj, al_list = _proj_call(src_in, Ws_l, vas, bsum)
        msgs = _edge_phase(xs_proj, al_list, srcs, dsts, z32h, zdh,
                           last=(l == 3))

    bsum_last = _bias_sums(b[2])
    p_zero, values = _mlp_heads(msgs[0], msgs[1], bsum_last[2], Wz0, bz0, Wz1, bz1,
                                Wz2, bz2, Wv0, bv0, Wv1, bv1, Wv2, bv2)
    zeros = jax.random.bernoulli(jax.random.key(1), p_zero).astype(jnp.float32)
    return (p_zero, zeros, values)
